# baseline (speedup 1.0000x reference)
import jax
import jax.numpy as jnp
from jax.experimental import pallas as pl

N = 64
N_IN = 128
N_H = 128
BB = 256


def _fused_gcn_kernel(adj_ref, seq_ref, w_ref, bias_ref, a_ref, out_ref):
    bb = out_ref.shape[0]
    seq = seq_ref[...].reshape(bb * N, N_IN)
    sf = jnp.dot(seq, w_ref[...], preferred_element_type=jnp.float32)
    sf = sf.reshape(bb, N, N_H).astype(jnp.bfloat16)
    out = jax.lax.dot_general(
        adj_ref[...].astype(jnp.bfloat16), sf,
        dimension_numbers=(((1,), (1,)), ((2,), (0,))),
        preferred_element_type=jnp.float32,
    )
    m = jnp.min(out, axis=1) + bias_ref[...]
    a = a_ref[0, 0]
    out_ref[...] = jnp.where(m >= 0, m, a * m)


def kernel(adj, seq1, W, bias, prelu_a):
    B = adj.shape[0]
    grid = (B // BB,)
    return pl.pallas_call(
        _fused_gcn_kernel,
        grid=grid,
        in_specs=[
            pl.BlockSpec((N - 1, N, BB), lambda i: (0, 0, i)),
            pl.BlockSpec((BB, N, N_IN), lambda i: (i, 0, 0)),
            pl.BlockSpec((N_IN, N_H), lambda i: (0, 0)),
            pl.BlockSpec((1, N_H), lambda i: (0, 0)),
            pl.BlockSpec((1, 1), lambda i: (0, 0)),
        ],
        out_specs=pl.BlockSpec((BB, N_H), lambda i: (i, 0)),
        out_shape=jax.ShapeDtypeStruct((B, N_H), jnp.float32),
    )(adj.transpose(1, 2, 0), seq1, W,
      bias.reshape(1, N_H), prelu_a.reshape(1, 1))
